# baseline (speedup 1.0000x reference)
import functools
import math

import jax
import jax.numpy as jnp
from jax import lax
from jax.experimental import pallas as pl
from jax.experimental.pallas import tpu as pltpu
from jax.experimental.pallas import tpu_sc as plsc

DIM = 1024
SCALE = math.sqrt(DIM)
LANES = 16

NUM_WORKERS = 32
CHUNK = 8
NGBUF = 4
NSBUF = 8


def kernel(x, table):
    batch, seq = x.shape
    n = batch * seq
    rows_per_worker = n // NUM_WORKERS
    nchunks = rows_per_worker // CHUNK
    assert nchunks % NSBUF == 0 and NSBUF % NGBUF == 0

    idx = x.reshape(n).astype(jnp.int32)
    mesh = plsc.VectorSubcoreMesh(
        core_axis_name="core", subcore_axis_name="subcore"
    )

    @functools.partial(
        pl.kernel,
        out_type=jax.ShapeDtypeStruct((n, DIM), jnp.float32),
        mesh=mesh,
        scratch_types=(
            [
                pltpu.VMEM((rows_per_worker,), jnp.int32),
                pltpu.VMEM((NGBUF, CHUNK, DIM), jnp.float32),
                pltpu.VMEM((NSBUF, CHUNK, DIM), jnp.float32),
            ]
            + [pltpu.SemaphoreType.DMA] * (NGBUF + NSBUF)
        ),
    )
    def emb_kernel(table_hbm, idx_hbm, out_hbm, idx_v, gbufs, sbufs, *sems):
        gsem = sems[:NGBUF]
        ssem = sems[NGBUF:]
        wid = lax.axis_index("subcore") * 2 + lax.axis_index("core")
        base = wid * rows_per_worker

        pltpu.sync_copy(idx_hbm.at[pl.ds(base, rows_per_worker)], idx_v)

        def gather_desc(j, b):
            return pltpu.make_async_copy(
                table_hbm.at[idx_v.at[pl.ds(j * CHUNK, CHUNK)]],
                gbufs.at[b],
                gsem[b],
            )

        def scatter_desc(j, b):
            return pltpu.make_async_copy(
                sbufs.at[b],
                out_hbm.at[pl.ds(base + j * CHUNK, CHUNK)],
                ssem[b],
            )

        for j0 in range(NGBUF - 1):
            gather_desc(j0, j0).start()

        @pl.loop(0, nchunks, step=NSBUF)
        def _(g):
            for b in range(NSBUF):
                j = g + b
                bg = b % NGBUF

                kg = j + NGBUF - 1
                bkg = (b + NGBUF - 1) % NGBUF

                @pl.when(kg < nchunks)
                def _():
                    gather_desc(kg, bkg).start()

                gather_desc(j, bg).wait()

                @pl.when(j >= NSBUF)
                def _():
                    scatter_desc(j - NSBUF, b).wait()

                @pl.loop(0, CHUNK)
                def _(r):
                    for v in range(DIM // LANES):
                        sl = pl.ds(v * LANES, LANES)
                        sbufs.at[b, r, sl][...] = (
                            gbufs.at[bg, r, sl][...] * SCALE
                        )

                scatter_desc(j, b).start()

        for b in range(NSBUF):
            scatter_desc(nchunks - NSBUF + b, b).wait()

    out = emb_kernel(table, idx)
    return out.reshape(batch, seq, DIM)
